# Initial kernel scaffold; baseline (speedup 1.0000x reference)
import jax
import jax.numpy as jnp
from jax.experimental import pallas as pl


def kernel(x, edge_index, edge_type, batch, shape_emb, color_emb, pos_emb, W1, root1, b1, W2, root2, b2, linW, linb):
    raise NotImplementedError("write your pallas kernel here")


# speedup vs baseline: 1.3692x; 1.3692x over previous
import jax
import jax.numpy as jnp
from jax import lax
from jax.experimental import pallas as pl
from jax.experimental.pallas import tpu as pltpu
from jax.experimental.pallas import tpu_sc as plsc

N = 10000
E = 320000
F = 128
R = 3
NGRAPH = 64
NCLASS = 10

NC = 2
NS = 16
NW = NC * NS

BLK = 128

NPAD = 10240
NPW = NPAD // NW
NH4 = NPAD // 4

EPAD = 327680
EPW = EPAD // NS
EBLK = EPW // BLK
NWIN = 4
WBLK = EBLK // NWIN

ECPW = EPAD // NW
CBLK = ECPW // BLK

VPW = 1024
VBLK = VPW // BLK

ACC_A = NPAD + 128
ACC_B = R * NH4 + 128
ACC_K = R * NPAD + 128
ZR_A = ACC_A // NS
ZR_B = ACC_B // NS
ZR_K = ACC_K // NS
WR_B = NH4 // NS
CW = 16

_MESH = plsc.VectorSubcoreMesh(core_axis_name="c", subcore_axis_name="s")


def _gs_pipeline(tbl, acc, gv, sv, rows0, rows1, s0, s1, nblk):
    @pl.loop(0, nblk)
    def _(j):
        pltpu.async_copy(tbl.at[gv.at[j]], rows0, s0).wait()
        pltpu.sync_copy(rows0, acc.at[sv.at[j]], add=True)


def _emb_body(tbl, gidx, sidx, zeros, out, acc, gv, sv, rows0, rows1, s0, s1):
    cid = lax.axis_index("c")
    sid = lax.axis_index("s")
    w = cid * NS + sid
    pltpu.sync_copy(zeros.at[pl.ds(sid * ZR_A, ZR_A)],
                    acc.at[pl.ds(sid * ZR_A, ZR_A)])
    plsc.subcore_barrier()

    pltpu.sync_copy(gidx.at[w], gv)
    pltpu.sync_copy(sidx.at[w], sv)
    _gs_pipeline(tbl, acc, gv, sv, rows0, rows1, s0, s1, VBLK)

    plsc.subcore_barrier()
    pltpu.sync_copy(acc.at[pl.ds(w * NPW, NPW)],
                    out.at[pl.ds(w * NPW, NPW)])


def _edge_body(hv, gidx, sidx, zeros, agg, acc, gv, sv, rows0, rows1, s0, s1):
    cid = lax.axis_index("c")
    sid = lax.axis_index("s")
    pltpu.sync_copy(zeros.at[pl.ds(sid * ZR_B, ZR_B)],
                    acc.at[pl.ds(sid * ZR_B, ZR_B)])
    plsc.subcore_barrier()
    for w in range(NWIN):
        pltpu.sync_copy(gidx.at[sid, pl.ds(w * WBLK, WBLK)], gv)
        pltpu.sync_copy(sidx.at[cid, sid, pl.ds(w * WBLK, WBLK)], sv)
        _gs_pipeline(hv, acc, gv, sv, rows0, rows1, s0, s1, WBLK)
    plsc.subcore_barrier()
    for r in range(R):
        pltpu.sync_copy(acc.at[pl.ds(r * NH4 + sid * WR_B, WR_B)],
                        agg.at[r, pl.ds(cid * NH4 + sid * WR_B, WR_B)])


def _edge_agg(hv, gidx, sidx, zeros):
    return pl.kernel(
        _edge_body,
        out_type=jax.ShapeDtypeStruct((R, NC * NH4, F), jnp.float32),
        mesh=_MESH,
        scratch_types=[
            pltpu.VMEM_SHARED((ACC_B, F), jnp.float32),
            pltpu.VMEM((WBLK, BLK), jnp.int32),
            pltpu.VMEM((WBLK, BLK), jnp.int32),
            pltpu.VMEM((BLK, F), jnp.float32),
            pltpu.VMEM((BLK, F), jnp.float32),
            pltpu.SemaphoreType.DMA,
            pltpu.SemaphoreType.DMA,
        ],
    )(hv, gidx, sidx, zeros)


def _combine_body(agg, cnt, h, W, root, b, out):
    x = h[...]
    acc = jnp.dot(x, root[...], preferred_element_type=jnp.float32) + b[...]
    for r in range(R):
        m = agg[r] / jnp.clip(cnt[r], 1.0)[:, None]
        acc = acc + jnp.dot(m, W[r], preferred_element_type=jnp.float32)
    out[...] = jnp.maximum(acc, 0.0)


_BR = 1024


def _combine(agg, cnt, h, W, root, b):
    grid = (NPAD // _BR,)
    return pl.pallas_call(
        _combine_body,
        grid=grid,
        in_specs=[
            pl.BlockSpec((R, _BR, F), lambda i: (0, i, 0)),
            pl.BlockSpec((R, _BR), lambda i: (0, i)),
            pl.BlockSpec((_BR, F), lambda i: (i, 0)),
            pl.BlockSpec((R, F, F), lambda i: (0, 0, 0)),
            pl.BlockSpec((F, F), lambda i: (0, 0)),
            pl.BlockSpec((1, F), lambda i: (0, 0)),
        ],
        out_specs=pl.BlockSpec((_BR, F), lambda i: (i, 0)),
        out_shape=jax.ShapeDtypeStruct((NPAD, F), jnp.float32),
    )(agg, cnt, h, W, root, b.reshape(1, F))


def _pool_body(h, b3, linW, linb, out, summ, cntg):
    i = pl.program_id(0)

    @pl.when(i == 0)
    def _():
        summ[...] = jnp.zeros_like(summ)
        cntg[...] = jnp.zeros_like(cntg)

    bb = b3[0, 0, :]
    gids = lax.broadcasted_iota(jnp.int32, (NGRAPH, _BR), 0)
    oh = (bb[None, :] == gids).astype(jnp.float32)
    summ[...] += jnp.dot(oh, h[...], preferred_element_type=jnp.float32)
    cntg[...] += jnp.sum(oh, axis=1, keepdims=True)

    @pl.when(i == pl.num_programs(0) - 1)
    def _():
        pooled = summ[...] / jnp.clip(cntg[...], 1.0)
        out[...] = (jnp.dot(pooled, linW[...],
                            preferred_element_type=jnp.float32) + linb[...])


def _pool(h2, batch3, linWp, linbp):
    grid = (NPAD // _BR,)
    return pl.pallas_call(
        _pool_body,
        grid=grid,
        in_specs=[
            pl.BlockSpec((_BR, F), lambda i: (i, 0)),
            pl.BlockSpec((1, 1, _BR), lambda i: (i, 0, 0)),
            pl.BlockSpec((F, F), lambda i: (0, 0)),
            pl.BlockSpec((1, F), lambda i: (0, 0)),
        ],
        out_specs=pl.BlockSpec((NGRAPH, F), lambda i: (0, 0)),
        out_shape=jax.ShapeDtypeStruct((NGRAPH, F), jnp.float32),
        scratch_shapes=[
            pltpu.VMEM((NGRAPH, F), jnp.float32),
            pltpu.VMEM((NGRAPH, F), jnp.float32),
        ],
    )(h2, batch3, linWp, linbp)


def kernel(x, edge_index, edge_type, batch, shape_emb, color_emb, pos_emb,
           W1, root1, b1, W2, root2, b2, linW, linb):
    i32 = jnp.int32
    f32 = jnp.float32

    x32 = x.astype(i32)
    tidx = jnp.stack([x32[:, 0], 16 + x32[:, 1], 32 + x32[:, 2]], axis=1)
    tidx = jnp.concatenate(
        [tidx, jnp.zeros((NPAD - N, 3), i32)], axis=0)
    vg = tidx.reshape(NW, 3 * NPW)
    vg = jnp.pad(vg, ((0, 0), (0, VPW - 3 * NPW)))
    gidx_a = vg.reshape(NW, VBLK, BLK)
    nodeid = jnp.repeat(jnp.arange(NPAD, dtype=i32), 3).reshape(NW, 3 * NPW)
    sidx_a = jnp.pad(nodeid, ((0, 0), (0, VPW - 3 * NPW)),
                     constant_values=NPAD)
    sidx_a = sidx_a.reshape(NW, VBLK, BLK)

    src = edge_index[0].astype(i32)
    dst = edge_index[1].astype(i32)
    et = edge_type.astype(i32)
    srcp = jnp.pad(src, (0, EPAD - E))
    gidx_e = srcp.reshape(NS, EBLK, BLK)
    dstp = jnp.pad(dst, (0, EPAD - E), constant_values=-1)
    etp = jnp.pad(et, (0, EPAD - E))

    def make_sidx(k):
        quarters = jnp.arange(NC, dtype=i32)[:, None] + 2 * k
        local = dstp[None, :] - quarters * NH4
        owned = (local >= 0) & (local < NH4)
        s = jnp.where(owned, etp[None, :] * NH4 + local, R * NH4)
        return s.astype(i32).reshape(NC, NS, EBLK, BLK)

    sidx_e0 = make_sidx(0)
    sidx_e1 = make_sidx(1)

    table = jnp.concatenate([shape_emb, color_emb, pos_emb], axis=0)
    zeros = jnp.zeros((ACC_A, F), f32)
    onesf = jnp.ones((NPAD, F), f32)

    batchp = jnp.concatenate(
        [batch.astype(i32), jnp.full((NPAD - N,), NGRAPH, i32)])
    batch3 = batchp.reshape(NPAD // _BR, 1, _BR)
    linWp = jnp.pad(linW, ((0, 0), (0, F - NCLASS)))
    linbp = jnp.pad(linb, (0, F - NCLASS)).reshape(1, F)

    h0 = pl.kernel(
        _emb_body,
        out_type=jax.ShapeDtypeStruct((NPAD, F), f32),
        mesh=_MESH,
        scratch_types=[
            pltpu.VMEM_SHARED((ACC_A, F), f32),
            pltpu.VMEM((VBLK, BLK), i32),
            pltpu.VMEM((VBLK, BLK), i32),
            pltpu.VMEM((BLK, F), f32),
            pltpu.VMEM((BLK, F), f32),
            pltpu.SemaphoreType.DMA,
            pltpu.SemaphoreType.DMA,
        ],
    )(table, gidx_a, sidx_a, zeros)

    c0 = _edge_agg(onesf, gidx_e, sidx_e0, zeros)
    c1 = _edge_agg(onesf, gidx_e, sidx_e1, zeros)
    cnt = jnp.concatenate([c0, c1], axis=1)[:, :, 0]

    a10 = _edge_agg(h0, gidx_e, sidx_e0, zeros)
    a11 = _edge_agg(h0, gidx_e, sidx_e1, zeros)
    agg1 = jnp.concatenate([a10, a11], axis=1)
    h1 = _combine(agg1, cnt, h0, W1, root1, b1)

    a20 = _edge_agg(h1, gidx_e, sidx_e0, zeros)
    a21 = _edge_agg(h1, gidx_e, sidx_e1, zeros)
    agg2 = jnp.concatenate([a20, a21], axis=1)
    h2 = _combine(agg2, cnt, h1, W2, root2, b2)

    outp = _pool(h2, batch3, linWp, linbp)
    return outp[:, :NCLASS]


# speedup vs baseline: 1.3861x; 1.0124x over previous
import jax
import jax.numpy as jnp
from jax import lax
from jax.experimental import pallas as pl
from jax.experimental.pallas import tpu as pltpu
from jax.experimental.pallas import tpu_sc as plsc

N = 10000
E = 320000
F = 128
R = 3
NGRAPH = 64
NCLASS = 10

NC = 2
NS = 16
NW = NC * NS

BLK = 128

NPAD = 10240
NPW = NPAD // NW
NH4 = NPAD // 4

EPAD = 327680
EPW = EPAD // NS
EBLK = EPW // BLK
NWIN = 4
WBLK = EBLK // NWIN

ECPW = EPAD // NW
CBLK = ECPW // BLK

VPW = 1024
VBLK = VPW // BLK

ACC_A = NPAD + 128
ACC_B = R * NH4 + 128
ACC_K = R * NPAD + 128
ZR_A = ACC_A // NS
ZR_B = ACC_B // NS
ZR_K = ACC_K // NS
WR_B = NH4 // NS
CW = 16

_MESH = plsc.VectorSubcoreMesh(core_axis_name="c", subcore_axis_name="s")


def _gs_pipeline(tbl, acc, gv, sv, rows0, rows1, s0, s1, nblk):
    pltpu.async_copy(tbl.at[gv.at[0]], rows0, s0)
    pltpu.async_copy(tbl.at[gv.at[1]], rows1, s1)

    @pl.loop(0, nblk - 2, step=2)
    def _(j):
        pltpu.make_async_copy(tbl.at[gv.at[0]], rows0, s0).wait()
        pltpu.sync_copy(rows0, acc.at[sv.at[j]], add=True)
        pltpu.async_copy(tbl.at[gv.at[j + 2]], rows0, s0)
        pltpu.make_async_copy(tbl.at[gv.at[0]], rows1, s1).wait()
        pltpu.sync_copy(rows1, acc.at[sv.at[j + 1]], add=True)
        pltpu.async_copy(tbl.at[gv.at[j + 3]], rows1, s1)

    pltpu.make_async_copy(tbl.at[gv.at[0]], rows0, s0).wait()
    pltpu.sync_copy(rows0, acc.at[sv.at[nblk - 2]], add=True)
    pltpu.make_async_copy(tbl.at[gv.at[0]], rows1, s1).wait()
    pltpu.sync_copy(rows1, acc.at[sv.at[nblk - 1]], add=True)


def _emb_body(tbl, gidx, sidx, zeros, out, acc, gv, sv, rows0, rows1, s0, s1):
    cid = lax.axis_index("c")
    sid = lax.axis_index("s")
    w = cid * NS + sid
    pltpu.sync_copy(zeros.at[pl.ds(sid * ZR_A, ZR_A)],
                    acc.at[pl.ds(sid * ZR_A, ZR_A)])
    plsc.subcore_barrier()

    pltpu.sync_copy(gidx.at[w], gv)
    pltpu.sync_copy(sidx.at[w], sv)
    _gs_pipeline(tbl, acc, gv, sv, rows0, rows1, s0, s1, VBLK)

    plsc.subcore_barrier()
    pltpu.sync_copy(acc.at[pl.ds(w * NPW, NPW)],
                    out.at[pl.ds(w * NPW, NPW)])


def _edge_body(hv, gidx, sidx, zeros, agg, acc, gv, sv, rows0, rows1, s0, s1):
    cid = lax.axis_index("c")
    sid = lax.axis_index("s")
    pltpu.sync_copy(zeros.at[pl.ds(sid * ZR_B, ZR_B)],
                    acc.at[pl.ds(sid * ZR_B, ZR_B)])
    plsc.subcore_barrier()
    for w in range(NWIN):
        pltpu.sync_copy(gidx.at[sid, pl.ds(w * WBLK, WBLK)], gv)
        pltpu.sync_copy(sidx.at[cid, sid, pl.ds(w * WBLK, WBLK)], sv)
        _gs_pipeline(hv, acc, gv, sv, rows0, rows1, s0, s1, WBLK)
    plsc.subcore_barrier()
    for r in range(R):
        pltpu.sync_copy(acc.at[pl.ds(r * NH4 + sid * WR_B, WR_B)],
                        agg.at[r, pl.ds(cid * NH4 + sid * WR_B, WR_B)])


def _edge_agg(hv, gidx, sidx, zeros):
    return pl.kernel(
        _edge_body,
        out_type=jax.ShapeDtypeStruct((R, NC * NH4, F), jnp.float32),
        mesh=_MESH,
        scratch_types=[
            pltpu.VMEM_SHARED((ACC_B, F), jnp.float32),
            pltpu.VMEM((WBLK, BLK), jnp.int32),
            pltpu.VMEM((WBLK, BLK), jnp.int32),
            pltpu.VMEM((BLK, F), jnp.float32),
            pltpu.VMEM((BLK, F), jnp.float32),
            pltpu.SemaphoreType.DMA,
            pltpu.SemaphoreType.DMA,
        ],
    )(hv, gidx, sidx, zeros)


def _combine_body(agg, cnt, h, W, root, b, out):
    x = h[...]
    acc = jnp.dot(x, root[...], preferred_element_type=jnp.float32) + b[...]
    for r in range(R):
        m = agg[r] / jnp.clip(cnt[r], 1.0)[:, None]
        acc = acc + jnp.dot(m, W[r], preferred_element_type=jnp.float32)
    out[...] = jnp.maximum(acc, 0.0)


_BR = 1024


def _combine(agg, cnt, h, W, root, b):
    grid = (NPAD // _BR,)
    return pl.pallas_call(
        _combine_body,
        grid=grid,
        in_specs=[
            pl.BlockSpec((R, _BR, F), lambda i: (0, i, 0)),
            pl.BlockSpec((R, _BR), lambda i: (0, i)),
            pl.BlockSpec((_BR, F), lambda i: (i, 0)),
            pl.BlockSpec((R, F, F), lambda i: (0, 0, 0)),
            pl.BlockSpec((F, F), lambda i: (0, 0)),
            pl.BlockSpec((1, F), lambda i: (0, 0)),
        ],
        out_specs=pl.BlockSpec((_BR, F), lambda i: (i, 0)),
        out_shape=jax.ShapeDtypeStruct((NPAD, F), jnp.float32),
    )(agg, cnt, h, W, root, b.reshape(1, F))


def _pool_body(h, b3, linW, linb, out, summ, cntg):
    i = pl.program_id(0)

    @pl.when(i == 0)
    def _():
        summ[...] = jnp.zeros_like(summ)
        cntg[...] = jnp.zeros_like(cntg)

    bb = b3[0, 0, :]
    gids = lax.broadcasted_iota(jnp.int32, (NGRAPH, _BR), 0)
    oh = (bb[None, :] == gids).astype(jnp.float32)
    summ[...] += jnp.dot(oh, h[...], preferred_element_type=jnp.float32)
    cntg[...] += jnp.sum(oh, axis=1, keepdims=True)

    @pl.when(i == pl.num_programs(0) - 1)
    def _():
        pooled = summ[...] / jnp.clip(cntg[...], 1.0)
        out[...] = (jnp.dot(pooled, linW[...],
                            preferred_element_type=jnp.float32) + linb[...])


def _pool(h2, batch3, linWp, linbp):
    grid = (NPAD // _BR,)
    return pl.pallas_call(
        _pool_body,
        grid=grid,
        in_specs=[
            pl.BlockSpec((_BR, F), lambda i: (i, 0)),
            pl.BlockSpec((1, 1, _BR), lambda i: (i, 0, 0)),
            pl.BlockSpec((F, F), lambda i: (0, 0)),
            pl.BlockSpec((1, F), lambda i: (0, 0)),
        ],
        out_specs=pl.BlockSpec((NGRAPH, F), lambda i: (0, 0)),
        out_shape=jax.ShapeDtypeStruct((NGRAPH, F), jnp.float32),
        scratch_shapes=[
            pltpu.VMEM((NGRAPH, F), jnp.float32),
            pltpu.VMEM((NGRAPH, F), jnp.float32),
        ],
    )(h2, batch3, linWp, linbp)


def kernel(x, edge_index, edge_type, batch, shape_emb, color_emb, pos_emb,
           W1, root1, b1, W2, root2, b2, linW, linb):
    i32 = jnp.int32
    f32 = jnp.float32

    x32 = x.astype(i32)
    tidx = jnp.stack([x32[:, 0], 16 + x32[:, 1], 32 + x32[:, 2]], axis=1)
    tidx = jnp.concatenate(
        [tidx, jnp.zeros((NPAD - N, 3), i32)], axis=0)
    vg = tidx.reshape(NW, 3 * NPW)
    vg = jnp.pad(vg, ((0, 0), (0, VPW - 3 * NPW)))
    gidx_a = vg.reshape(NW, VBLK, BLK)
    nodeid = jnp.repeat(jnp.arange(NPAD, dtype=i32), 3).reshape(NW, 3 * NPW)
    sidx_a = jnp.pad(nodeid, ((0, 0), (0, VPW - 3 * NPW)),
                     constant_values=NPAD)
    sidx_a = sidx_a.reshape(NW, VBLK, BLK)

    src = edge_index[0].astype(i32)
    dst = edge_index[1].astype(i32)
    et = edge_type.astype(i32)
    srcp = jnp.pad(src, (0, EPAD - E))
    gidx_e = srcp.reshape(NS, EBLK, BLK)
    dstp = jnp.pad(dst, (0, EPAD - E), constant_values=-1)
    etp = jnp.pad(et, (0, EPAD - E))

    def make_sidx(k):
        quarters = jnp.arange(NC, dtype=i32)[:, None] + 2 * k
        local = dstp[None, :] - quarters * NH4
        owned = (local >= 0) & (local < NH4)
        s = jnp.where(owned, etp[None, :] * NH4 + local, R * NH4)
        return s.astype(i32).reshape(NC, NS, EBLK, BLK)

    sidx_e0 = make_sidx(0)
    sidx_e1 = make_sidx(1)

    table = jnp.concatenate([shape_emb, color_emb, pos_emb], axis=0)
    zeros = jnp.zeros((ACC_A, F), f32)
    onesf = jnp.ones((NPAD, F), f32)

    batchp = jnp.concatenate(
        [batch.astype(i32), jnp.full((NPAD - N,), NGRAPH, i32)])
    batch3 = batchp.reshape(NPAD // _BR, 1, _BR)
    linWp = jnp.pad(linW, ((0, 0), (0, F - NCLASS)))
    linbp = jnp.pad(linb, (0, F - NCLASS)).reshape(1, F)

    h0 = pl.kernel(
        _emb_body,
        out_type=jax.ShapeDtypeStruct((NPAD, F), f32),
        mesh=_MESH,
        scratch_types=[
            pltpu.VMEM_SHARED((ACC_A, F), f32),
            pltpu.VMEM((VBLK, BLK), i32),
            pltpu.VMEM((VBLK, BLK), i32),
            pltpu.VMEM((BLK, F), f32),
            pltpu.VMEM((BLK, F), f32),
            pltpu.SemaphoreType.DMA,
            pltpu.SemaphoreType.DMA,
        ],
    )(table, gidx_a, sidx_a, zeros)

    c0 = _edge_agg(onesf, gidx_e, sidx_e0, zeros)
    c1 = _edge_agg(onesf, gidx_e, sidx_e1, zeros)
    cnt = jnp.concatenate([c0, c1], axis=1)[:, :, 0]

    a10 = _edge_agg(h0, gidx_e, sidx_e0, zeros)
    a11 = _edge_agg(h0, gidx_e, sidx_e1, zeros)
    agg1 = jnp.concatenate([a10, a11], axis=1)
    h1 = _combine(agg1, cnt, h0, W1, root1, b1)

    a20 = _edge_agg(h1, gidx_e, sidx_e0, zeros)
    a21 = _edge_agg(h1, gidx_e, sidx_e1, zeros)
    agg2 = jnp.concatenate([a20, a21], axis=1)
    h2 = _combine(agg2, cnt, h1, W2, root2, b2)

    outp = _pool(h2, batch3, linWp, linbp)
    return outp[:, :NCLASS]


# speedup vs baseline: 1.7412x; 1.2562x over previous
import jax
import jax.numpy as jnp
from jax import lax
from jax.experimental import pallas as pl
from jax.experimental.pallas import tpu as pltpu
from jax.experimental.pallas import tpu_sc as plsc

N = 10000
E = 320000
F = 128
R = 3
NGRAPH = 64
NCLASS = 10

NC = 2
NS = 16
NW = NC * NS

BLK = 128

NPAD = 10240
NPW = NPAD // NW
NH4 = NPAD // 4

EPAD = 327680
EPW = EPAD // NS
EBLK = EPW // BLK
NWIN = 4
WBLK = EBLK // NWIN

ECPW = EPAD // NW
CBLK = ECPW // BLK

VPW = 1024
VBLK = VPW // BLK

ACC_A = NPAD + 128
ACC_B = R * NH4 + 128
ACC_K = R * NPAD + 128
ZR_A = ACC_A // NS
ZR_B = ACC_B // NS
ZR_K = ACC_K // NS
WR_B = NH4 // NS
CW = 16

_MESH = plsc.VectorSubcoreMesh(core_axis_name="c", subcore_axis_name="s")


def _gs_pipeline(tbl, acc, gv, sv, rows0, rows1, s0, s1, nblk):
    pltpu.async_copy(tbl.at[gv.at[0]], rows0, s0)
    pltpu.async_copy(tbl.at[gv.at[1]], rows1, s1)

    @pl.loop(0, nblk - 2, step=2)
    def _(j):
        pltpu.make_async_copy(tbl.at[gv.at[0]], rows0, s0).wait()
        pltpu.sync_copy(rows0, acc.at[sv.at[j]], add=True)
        pltpu.async_copy(tbl.at[gv.at[j + 2]], rows0, s0)
        pltpu.make_async_copy(tbl.at[gv.at[0]], rows1, s1).wait()
        pltpu.sync_copy(rows1, acc.at[sv.at[j + 1]], add=True)
        pltpu.async_copy(tbl.at[gv.at[j + 3]], rows1, s1)

    pltpu.make_async_copy(tbl.at[gv.at[0]], rows0, s0).wait()
    pltpu.sync_copy(rows0, acc.at[sv.at[nblk - 2]], add=True)
    pltpu.make_async_copy(tbl.at[gv.at[0]], rows1, s1).wait()
    pltpu.sync_copy(rows1, acc.at[sv.at[nblk - 1]], add=True)


def _emb_body(tbl, gidx, sidx, zeros, out, acc, gv, sv, rows0, rows1, s0, s1):
    cid = lax.axis_index("c")
    sid = lax.axis_index("s")
    w = cid * NS + sid
    pltpu.sync_copy(zeros.at[pl.ds(sid * ZR_A, ZR_A)],
                    acc.at[pl.ds(sid * ZR_A, ZR_A)])
    plsc.subcore_barrier()

    pltpu.sync_copy(gidx.at[w], gv)
    pltpu.sync_copy(sidx.at[w], sv)
    _gs_pipeline(tbl, acc, gv, sv, rows0, rows1, s0, s1, VBLK)

    plsc.subcore_barrier()
    pltpu.sync_copy(acc.at[pl.ds(w * NPW, NPW)],
                    out.at[pl.ds(w * NPW, NPW)])


def _cnt_body(sidx, zeros, ones_h, cnt, acc, sv, onesv):
    cid = lax.axis_index("c")
    sid = lax.axis_index("s")
    pltpu.sync_copy(zeros.at[pl.ds(sid * ZR_B, ZR_B)],
                    acc.at[pl.ds(sid * ZR_B, ZR_B)])
    pltpu.sync_copy(ones_h, onesv)
    plsc.subcore_barrier()
    pltpu.sync_copy(sidx.at[cid, sid], sv)

    @pl.loop(0, EBLK)
    def _(j):
        pltpu.sync_copy(onesv, acc.at[sv.at[j]], add=True)

    plsc.subcore_barrier()
    for r in range(R):
        pltpu.sync_copy(acc.at[pl.ds(r * NH4 + sid * WR_B, WR_B)],
                        cnt.at[r, pl.ds(cid * NH4 + sid * WR_B, WR_B)])


def _cnt_agg(sidx, zeros, ones):
    return pl.kernel(
        _cnt_body,
        out_type=jax.ShapeDtypeStruct((R, NC * NH4, F), jnp.float32),
        mesh=_MESH,
        scratch_types=[
            pltpu.VMEM_SHARED((ACC_B, F), jnp.float32),
            pltpu.VMEM((EBLK, BLK), jnp.int32),
            pltpu.VMEM((BLK, F), jnp.float32),
        ],
    )(sidx, zeros, ones)


def _edge_body(hv, gidx, sidx, zeros, agg, acc, gv, sv, rows0, rows1, s0, s1):
    cid = lax.axis_index("c")
    sid = lax.axis_index("s")
    pltpu.sync_copy(zeros.at[pl.ds(sid * ZR_B, ZR_B)],
                    acc.at[pl.ds(sid * ZR_B, ZR_B)])
    plsc.subcore_barrier()
    for w in range(NWIN):
        pltpu.sync_copy(gidx.at[sid, pl.ds(w * WBLK, WBLK)], gv)
        pltpu.sync_copy(sidx.at[cid, sid, pl.ds(w * WBLK, WBLK)], sv)
        _gs_pipeline(hv, acc, gv, sv, rows0, rows1, s0, s1, WBLK)
    plsc.subcore_barrier()
    for r in range(R):
        pltpu.sync_copy(acc.at[pl.ds(r * NH4 + sid * WR_B, WR_B)],
                        agg.at[r, pl.ds(cid * NH4 + sid * WR_B, WR_B)])


def _edge_agg(hv, gidx, sidx, zeros):
    return pl.kernel(
        _edge_body,
        out_type=jax.ShapeDtypeStruct((R, NC * NH4, F), jnp.float32),
        mesh=_MESH,
        scratch_types=[
            pltpu.VMEM_SHARED((ACC_B, F), jnp.float32),
            pltpu.VMEM((WBLK, BLK), jnp.int32),
            pltpu.VMEM((WBLK, BLK), jnp.int32),
            pltpu.VMEM((BLK, F), jnp.float32),
            pltpu.VMEM((BLK, F), jnp.float32),
            pltpu.SemaphoreType.DMA,
            pltpu.SemaphoreType.DMA,
        ],
    )(hv, gidx, sidx, zeros)


def _combine_body(agg, cnt, h, W, root, b, out):
    x = h[...]
    acc = jnp.dot(x, root[...], preferred_element_type=jnp.float32) + b[...]
    for r in range(R):
        m = agg[r] / jnp.clip(cnt[r], 1.0)[:, None]
        acc = acc + jnp.dot(m, W[r], preferred_element_type=jnp.float32)
    out[...] = jnp.maximum(acc, 0.0)


_BR = 1024


def _combine(agg, cnt, h, W, root, b):
    grid = (NPAD // _BR,)
    return pl.pallas_call(
        _combine_body,
        grid=grid,
        in_specs=[
            pl.BlockSpec((R, _BR, F), lambda i: (0, i, 0)),
            pl.BlockSpec((R, _BR), lambda i: (0, i)),
            pl.BlockSpec((_BR, F), lambda i: (i, 0)),
            pl.BlockSpec((R, F, F), lambda i: (0, 0, 0)),
            pl.BlockSpec((F, F), lambda i: (0, 0)),
            pl.BlockSpec((1, F), lambda i: (0, 0)),
        ],
        out_specs=pl.BlockSpec((_BR, F), lambda i: (i, 0)),
        out_shape=jax.ShapeDtypeStruct((NPAD, F), jnp.float32),
    )(agg, cnt, h, W, root, b.reshape(1, F))


def _pool_body(h, b3, linW, linb, out, summ, cntg):
    i = pl.program_id(0)

    @pl.when(i == 0)
    def _():
        summ[...] = jnp.zeros_like(summ)
        cntg[...] = jnp.zeros_like(cntg)

    bb = b3[0, 0, :]
    gids = lax.broadcasted_iota(jnp.int32, (NGRAPH, _BR), 0)
    oh = (bb[None, :] == gids).astype(jnp.float32)
    summ[...] += jnp.dot(oh, h[...], preferred_element_type=jnp.float32)
    cntg[...] += jnp.sum(oh, axis=1, keepdims=True)

    @pl.when(i == pl.num_programs(0) - 1)
    def _():
        pooled = summ[...] / jnp.clip(cntg[...], 1.0)
        out[...] = (jnp.dot(pooled, linW[...],
                            preferred_element_type=jnp.float32) + linb[...])


def _pool(h2, batch3, linWp, linbp):
    grid = (NPAD // _BR,)
    return pl.pallas_call(
        _pool_body,
        grid=grid,
        in_specs=[
            pl.BlockSpec((_BR, F), lambda i: (i, 0)),
            pl.BlockSpec((1, 1, _BR), lambda i: (i, 0, 0)),
            pl.BlockSpec((F, F), lambda i: (0, 0)),
            pl.BlockSpec((1, F), lambda i: (0, 0)),
        ],
        out_specs=pl.BlockSpec((NGRAPH, F), lambda i: (0, 0)),
        out_shape=jax.ShapeDtypeStruct((NGRAPH, F), jnp.float32),
        scratch_shapes=[
            pltpu.VMEM((NGRAPH, F), jnp.float32),
            pltpu.VMEM((NGRAPH, F), jnp.float32),
        ],
    )(h2, batch3, linWp, linbp)


def kernel(x, edge_index, edge_type, batch, shape_emb, color_emb, pos_emb,
           W1, root1, b1, W2, root2, b2, linW, linb):
    i32 = jnp.int32
    f32 = jnp.float32

    x32 = x.astype(i32)
    tidx = jnp.stack([x32[:, 0], 16 + x32[:, 1], 32 + x32[:, 2]], axis=1)
    tidx = jnp.concatenate(
        [tidx, jnp.zeros((NPAD - N, 3), i32)], axis=0)
    vg = tidx.reshape(NW, 3 * NPW)
    vg = jnp.pad(vg, ((0, 0), (0, VPW - 3 * NPW)))
    gidx_a = vg.reshape(NW, VBLK, BLK)
    nodeid = jnp.repeat(jnp.arange(NPAD, dtype=i32), 3).reshape(NW, 3 * NPW)
    sidx_a = jnp.pad(nodeid, ((0, 0), (0, VPW - 3 * NPW)),
                     constant_values=NPAD)
    sidx_a = sidx_a.reshape(NW, VBLK, BLK)

    src = edge_index[0].astype(i32)
    dst = edge_index[1].astype(i32)
    et = edge_type.astype(i32)
    srcp = jnp.pad(src, (0, EPAD - E))
    gidx_e = srcp.reshape(NS, EBLK, BLK)
    dstp = jnp.pad(dst, (0, EPAD - E), constant_values=-1)
    etp = jnp.pad(et, (0, EPAD - E))

    def make_sidx(k):
        quarters = jnp.arange(NC, dtype=i32)[:, None] + 2 * k
        local = dstp[None, :] - quarters * NH4
        owned = (local >= 0) & (local < NH4)
        s = jnp.where(owned, etp[None, :] * NH4 + local, R * NH4)
        return s.astype(i32).reshape(NC, NS, EBLK, BLK)

    sidx_e0 = make_sidx(0)
    sidx_e1 = make_sidx(1)

    table = jnp.concatenate([shape_emb, color_emb, pos_emb], axis=0)
    zeros = jnp.zeros((ACC_A, F), f32)
    ones = jnp.ones((BLK, F), f32)

    batchp = jnp.concatenate(
        [batch.astype(i32), jnp.full((NPAD - N,), NGRAPH, i32)])
    batch3 = batchp.reshape(NPAD // _BR, 1, _BR)
    linWp = jnp.pad(linW, ((0, 0), (0, F - NCLASS)))
    linbp = jnp.pad(linb, (0, F - NCLASS)).reshape(1, F)

    h0 = pl.kernel(
        _emb_body,
        out_type=jax.ShapeDtypeStruct((NPAD, F), f32),
        mesh=_MESH,
        scratch_types=[
            pltpu.VMEM_SHARED((ACC_A, F), f32),
            pltpu.VMEM((VBLK, BLK), i32),
            pltpu.VMEM((VBLK, BLK), i32),
            pltpu.VMEM((BLK, F), f32),
            pltpu.VMEM((BLK, F), f32),
            pltpu.SemaphoreType.DMA,
            pltpu.SemaphoreType.DMA,
        ],
    )(table, gidx_a, sidx_a, zeros)

    c0 = _cnt_agg(sidx_e0, zeros, ones)
    c1 = _cnt_agg(sidx_e1, zeros, ones)
    cnt = jnp.concatenate([c0, c1], axis=1)[:, :, 0]

    a10 = _edge_agg(h0, gidx_e, sidx_e0, zeros)
    a11 = _edge_agg(h0, gidx_e, sidx_e1, zeros)
    agg1 = jnp.concatenate([a10, a11], axis=1)
    h1 = _combine(agg1, cnt, h0, W1, root1, b1)

    a20 = _edge_agg(h1, gidx_e, sidx_e0, zeros)
    a21 = _edge_agg(h1, gidx_e, sidx_e1, zeros)
    agg2 = jnp.concatenate([a20, a21], axis=1)
    h2 = _combine(agg2, cnt, h1, W2, root2, b2)

    outp = _pool(h2, batch3, linWp, linbp)
    return outp[:, :NCLASS]
